# baseline (speedup 1.0000x reference)
import functools

import jax
import jax.numpy as jnp
import numpy as np
from jax import lax
from jax.experimental import pallas as pl
from jax.experimental.pallas import tpu as pltpu
from jax.experimental.pallas import tpu_sc as plsc

_B = 64
_C = 3
_H = 128
_W = 128
_Q = 512
_K = 21

_D = _C * _H * _W
_KD = 441
_KDP = 512

_NW = 32
_KA_W = (_Q - _B) // 16

_IDX = np.array([
    121, 480, 35, 130, 263, 148, 197, 410, 398, 45, 176, 462, 446, 366, 257,
    179, 139, 315, 501, 188, 312, 499, 318, 448, 304, 99, 309, 144, 152, 189,
    487, 325, 31, 112, 495, 356, 493, 507, 268, 429, 409, 85, 63, 117, 417,
    174, 441, 509, 481, 272, 114, 254, 82, 65, 7, 350, 4, 101, 463, 452, 444,
    102, 78, 163, 157, 302, 183, 29, 240, 177, 278, 259, 108, 305, 83, 129,
    367, 212, 277, 504, 300, 44, 211, 16, 58, 123, 37, 336, 111, 19, 61, 447,
    2, 142, 34, 369, 339, 156, 436, 5, 461, 415, 90, 363, 175, 167, 284, 379,
    251, 110, 72, 155, 178, 323, 291, 388, 269, 354, 368, 219, 510, 153, 30,
    275, 42, 186, 342, 406, 468, 439, 307, 256, 419, 246, 3, 362, 380, 327,
    393, 70, 378, 400, 271, 488, 311, 67, 273, 223, 422, 39, 56, 274, 192,
    169, 349, 218, 195, 476, 173, 245, 241, 69, 383, 80, 22, 6, 321, 199, 345,
    118, 235, 54, 442, 479, 423, 266, 77, 425, 147, 18, 340, 298, 249, 294,
    375, 382, 10, 11, 234, 53, 236, 455, 94, 332, 511, 331, 437, 353, 489,
    287, 32, 217, 283, 355, 407, 159, 440, 15, 470, 184, 49, 137, 50, 138, 20,
    445, 237, 280, 253, 185, 460, 43, 389, 335, 258, 370, 344, 92, 8, 503,
    324, 140, 233, 24, 81, 239, 314, 453, 96, 475, 467, 154, 135, 472, 490,
    469, 500, 264, 160, 106, 128, 265, 426, 386, 191, 9, 200, 40, 187, 71,
    346, 438, 333, 248, 164, 207, 93, 59, 201, 158, 210, 420, 402, 75, 508,
    131, 411, 97, 66, 25, 196, 424, 364, 497, 242, 338, 206, 243, 397, 341,
    450, 414, 238, 295, 432, 431, 308, 73, 320, 13, 52, 491, 203, 289, 303,
    202, 255, 194, 88, 250, 337, 62, 230, 150, 261, 330, 262, 209, 132, 357,
    87, 76, 198, 486, 60, 244, 457, 47, 392, 374, 276, 33, 79, 451, 180, 403,
    247, 14, 459, 286, 421, 458, 228, 17, 38, 86, 231, 190, 232, 482, 23, 105,
    484, 395, 427, 301, 474, 376, 405, 494, 471, 391, 313, 220, 0, 473, 145,
    371, 213, 226, 381, 133, 281, 41, 64, 416, 21, 443, 161, 279, 285, 166,
    124, 116, 449, 26, 165, 168, 193, 57, 208, 181, 89, 146, 182, 126, 125,
    297, 1, 115, 28, 113, 225, 361, 351, 465, 172, 377, 162, 48, 170, 466,
    505, 227, 36, 252, 502, 492, 119, 151, 385, 306, 120, 372, 390, 224, 122,
    270, 100, 418, 433, 329, 365, 396, 91, 222, 55, 496, 498, 103, 51, 293,
    215, 384, 127, 98, 483, 506, 282, 107, 27, 322, 74, 136, 229, 319, 328,
    430, 343, 204, 221, 296, 12, 134, 454, 477, 408, 109, 84, 428, 317, 358,
    394, 299, 205, 171, 288, 143, 68, 267, 216, 435, 149, 485, 434, 141, 464,
    334, 404, 104, 352, 95, 387, 316, 214, 290, 46, 310, 348, 401, 260, 478,
    292, 359, 326, 347, 456, 399, 373, 412, 360, 413], dtype=np.int64)

_KIDX = np.concatenate([_IDX[_B:], _IDX[:_B]]).astype(np.int32)

_INV = np.argsort(_IDX)

_mesh = plsc.VectorSubcoreMesh(core_axis_name="c", subcore_axis_name="s")

_img_out = [
    jax.ShapeDtypeStruct((_Q, _C, _H, _W), jnp.float32),
    jax.ShapeDtypeStruct((_B, _C, _H, _W), jnp.float32),
]

_ROWS_W = _Q // _NW
_HEAD_W = _B // _NW


@functools.partial(pl.kernel, out_type=_img_out, mesh=_mesh,
                   scratch_types=[
                       pltpu.VMEM_SHARED((16, 2, _C, _H, _W), jnp.float32),
                       pltpu.SemaphoreType.DMA,
                       pltpu.SemaphoreType.DMA])
def _sc_img_stream(tbl, batch, newt, deqt, slots, lsem, wsem):
    w = lax.axis_index("s") * 2 + lax.axis_index("c")

    def worker_prog(wi):
        sid = wi // 2
        tasks = []
        for s in range(wi * _ROWS_W, (wi + 1) * _ROWS_W):
            j = int(_INV[s])
            if j < _B:
                tasks.append((tbl, s, deqt, j))
            else:
                tasks.append((tbl, s, newt, j))
        for r in range(wi * _HEAD_W, (wi + 1) * _HEAD_W):
            tasks.append((batch, r, newt, r))

        n = len(tasks)
        hl = [None] * n
        hw = [None] * n
        for i in range(n + 1):
            if i < n:
                if i >= 2:
                    hw[i - 2].wait()
                src_ref, s, _, _ = tasks[i]
                hl[i] = pltpu.async_copy(src_ref.at[s],
                                         slots.at[sid, i % 2], lsem)
            if i >= 1:
                _, _, dst_ref, j = tasks[i - 1]
                hl[i - 1].wait()
                hw[i - 1] = pltpu.async_copy(slots.at[sid, (i - 1) % 2],
                                             dst_ref.at[j], wsem)
        hw[n - 2].wait()
        hw[n - 1].wait()

    for wi in range(_NW):
        @pl.when(w == wi)
        def _(wi=wi):
            worker_prog(wi)


@functools.partial(
    pl.kernel,
    out_type=[
        jax.ShapeDtypeStruct((_Q, _KDP), jnp.float32),
        jax.ShapeDtypeStruct((_B, _KDP), jnp.float32),
    ],
    mesh=_mesh,
    scratch_types=[
        pltpu.VMEM((16,), jnp.int32),
        pltpu.VMEM((16, _KDP), jnp.float32),
        pltpu.SemaphoreType.DMA,
    ],
)
def _sc_ker_stream(kidx, ker2, lr2, newker2, deqker2, kidx_v, kbuf, hsem):
    w = lax.axis_index("s") * 2 + lax.axis_index("c")

    h0 = pltpu.async_copy(lr2.at[pl.ds(w * 2, 2)],
                          newker2.at[pl.ds(w * 2, 2)], hsem)

    def ker_chunk(idx_off, dst, dst_off):
        pltpu.sync_copy(kidx.at[pl.ds(idx_off, 16)], kidx_v)
        pltpu.async_copy(ker2.at[kidx_v], kbuf, hsem).wait()
        pltpu.sync_copy(kbuf, dst.at[pl.ds(dst_off, 16)])

    @pl.when(w < _KA_W)
    def _():
        ker_chunk(w * 16, newker2, _B + w * 16)

    @pl.when(w >= _KA_W)
    def _():
        ker_chunk((_Q - _B) + (w - _KA_W) * 16, deqker2, (w - _KA_W) * 16)

    h0.wait()


_T = _Q + _B
_SRCQ = np.empty((_T,), np.int32)
_SRCQ[:_B] = _IDX[:_B]
_SRCQ[_B:2 * _B] = _IDX[_B]
_SRCQ[2 * _B:] = _IDX[_B:]
_SRCB = np.empty((_T,), np.int32)
_SRCB[:_B] = 0
_SRCB[_B:2 * _B] = np.arange(_B)
_SRCB[2 * _B:] = _B - 1
_DSTN = np.empty((_T,), np.int32)
_DSTN[:_B] = 0
_DSTN[_B:] = np.arange(_T - _B)
_DSTD = np.empty((_T,), np.int32)
_DSTD[:_B] = np.arange(_B)
_DSTD[_B:] = _B - 1
_TCMAPS = np.stack([_SRCQ, _SRCB, _DSTN, _DSTD])


def _tc_body(maps_ref, tbl_ref, batch_ref, new_ref, deq_ref):
    t = pl.program_id(0)

    @pl.when(t < _B)
    def _():
        deq_ref[...] = tbl_ref[...]

    @pl.when(jnp.logical_and(t >= _B, t < 2 * _B))
    def _():
        new_ref[...] = batch_ref[...]

    @pl.when(t >= 2 * _B)
    def _():
        new_ref[...] = tbl_ref[...]


_tc_img_stream = pl.pallas_call(
    _tc_body,
    grid_spec=pltpu.PrefetchScalarGridSpec(
        num_scalar_prefetch=1,
        grid=(_T,),
        in_specs=[
            pl.BlockSpec((1, _C, _H, _W),
                         lambda t, m: (m[0, t], 0, 0, 0)),
            pl.BlockSpec((1, _C, _H, _W),
                         lambda t, m: (m[1, t], 0, 0, 0)),
        ],
        out_specs=[
            pl.BlockSpec((1, _C, _H, _W),
                         lambda t, m: (m[2, t], 0, 0, 0)),
            pl.BlockSpec((1, _C, _H, _W),
                         lambda t, m: (m[3, t], 0, 0, 0)),
        ],
    ),
    out_shape=[
        jax.ShapeDtypeStruct((_Q, _C, _H, _W), jnp.float32),
        jax.ShapeDtypeStruct((_B, _C, _H, _W), jnp.float32),
    ],
)


def kernel(query, key_img, lr_gt_kernel, queue_q, queue_k, queue_ker):
    ker2 = jnp.pad(queue_ker.reshape(_Q, _KD), ((0, 0), (0, _KDP - _KD)))
    lr2 = jnp.pad(lr_gt_kernel.reshape(_B, _KD), ((0, 0), (0, _KDP - _KD)))
    kidx = jnp.asarray(_KIDX)
    tcmaps = jnp.asarray(_TCMAPS)

    new_qk, k_deq = _sc_img_stream(queue_k, key_img)
    newker2, deqker2 = _sc_ker_stream(kidx, ker2, lr2)
    new_qq, q_deq = _tc_img_stream(tcmaps, queue_q, query)

    new_qker = newker2[:, :_KD].reshape(_Q, 1, _K, _K)
    ker_deq = deqker2[:, :_KD].reshape(_B, 1, _K, _K)
    return (q_deq, k_deq, ker_deq, new_qq, new_qk, new_qker)


# speedup vs baseline: 1.9620x; 1.9620x over previous
import functools

import jax
import jax.numpy as jnp
import numpy as np
from jax import lax
from jax.experimental import pallas as pl
from jax.experimental.pallas import tpu as pltpu
from jax.experimental.pallas import tpu_sc as plsc

_B = 64
_C = 3
_H = 128
_W = 128
_Q = 512
_K = 21

_D = _C * _H * _W
_KD = 441
_KDP = 512

_NW = 32
_KA_W = (_Q - _B) // 16

_IDX = np.array([
    121, 480, 35, 130, 263, 148, 197, 410, 398, 45, 176, 462, 446, 366, 257,
    179, 139, 315, 501, 188, 312, 499, 318, 448, 304, 99, 309, 144, 152, 189,
    487, 325, 31, 112, 495, 356, 493, 507, 268, 429, 409, 85, 63, 117, 417,
    174, 441, 509, 481, 272, 114, 254, 82, 65, 7, 350, 4, 101, 463, 452, 444,
    102, 78, 163, 157, 302, 183, 29, 240, 177, 278, 259, 108, 305, 83, 129,
    367, 212, 277, 504, 300, 44, 211, 16, 58, 123, 37, 336, 111, 19, 61, 447,
    2, 142, 34, 369, 339, 156, 436, 5, 461, 415, 90, 363, 175, 167, 284, 379,
    251, 110, 72, 155, 178, 323, 291, 388, 269, 354, 368, 219, 510, 153, 30,
    275, 42, 186, 342, 406, 468, 439, 307, 256, 419, 246, 3, 362, 380, 327,
    393, 70, 378, 400, 271, 488, 311, 67, 273, 223, 422, 39, 56, 274, 192,
    169, 349, 218, 195, 476, 173, 245, 241, 69, 383, 80, 22, 6, 321, 199, 345,
    118, 235, 54, 442, 479, 423, 266, 77, 425, 147, 18, 340, 298, 249, 294,
    375, 382, 10, 11, 234, 53, 236, 455, 94, 332, 511, 331, 437, 353, 489,
    287, 32, 217, 283, 355, 407, 159, 440, 15, 470, 184, 49, 137, 50, 138, 20,
    445, 237, 280, 253, 185, 460, 43, 389, 335, 258, 370, 344, 92, 8, 503,
    324, 140, 233, 24, 81, 239, 314, 453, 96, 475, 467, 154, 135, 472, 490,
    469, 500, 264, 160, 106, 128, 265, 426, 386, 191, 9, 200, 40, 187, 71,
    346, 438, 333, 248, 164, 207, 93, 59, 201, 158, 210, 420, 402, 75, 508,
    131, 411, 97, 66, 25, 196, 424, 364, 497, 242, 338, 206, 243, 397, 341,
    450, 414, 238, 295, 432, 431, 308, 73, 320, 13, 52, 491, 203, 289, 303,
    202, 255, 194, 88, 250, 337, 62, 230, 150, 261, 330, 262, 209, 132, 357,
    87, 76, 198, 486, 60, 244, 457, 47, 392, 374, 276, 33, 79, 451, 180, 403,
    247, 14, 459, 286, 421, 458, 228, 17, 38, 86, 231, 190, 232, 482, 23, 105,
    484, 395, 427, 301, 474, 376, 405, 494, 471, 391, 313, 220, 0, 473, 145,
    371, 213, 226, 381, 133, 281, 41, 64, 416, 21, 443, 161, 279, 285, 166,
    124, 116, 449, 26, 165, 168, 193, 57, 208, 181, 89, 146, 182, 126, 125,
    297, 1, 115, 28, 113, 225, 361, 351, 465, 172, 377, 162, 48, 170, 466,
    505, 227, 36, 252, 502, 492, 119, 151, 385, 306, 120, 372, 390, 224, 122,
    270, 100, 418, 433, 329, 365, 396, 91, 222, 55, 496, 498, 103, 51, 293,
    215, 384, 127, 98, 483, 506, 282, 107, 27, 322, 74, 136, 229, 319, 328,
    430, 343, 204, 221, 296, 12, 134, 454, 477, 408, 109, 84, 428, 317, 358,
    394, 299, 205, 171, 288, 143, 68, 267, 216, 435, 149, 485, 434, 141, 464,
    334, 404, 104, 352, 95, 387, 316, 214, 290, 46, 310, 348, 401, 260, 478,
    292, 359, 326, 347, 456, 399, 373, 412, 360, 413], dtype=np.int64)

_KIDX = np.concatenate([_IDX[_B:], _IDX[:_B]]).astype(np.int32)

_INV = np.argsort(_IDX)

_mesh = plsc.VectorSubcoreMesh(core_axis_name="c", subcore_axis_name="s")

_img_out = [
    jax.ShapeDtypeStruct((_Q, _C, _H, _W), jnp.float32),
    jax.ShapeDtypeStruct((_B, _C, _H, _W), jnp.float32),
]

_ROWS_W = _Q // _NW
_HEAD_W = _B // _NW


@functools.partial(pl.kernel, out_type=_img_out, mesh=_mesh,
                   scratch_types=[
                       pltpu.VMEM_SHARED((16, 2, _C, _H, _W), jnp.float32),
                       pltpu.SemaphoreType.DMA,
                       pltpu.SemaphoreType.DMA])
def _sc_img_stream(tbl, batch, newt, deqt, slots, lsem, wsem):
    w = lax.axis_index("s") * 2 + lax.axis_index("c")

    def worker_prog(wi):
        sid = wi // 2
        tasks = []
        for s in range(wi * _ROWS_W, (wi + 1) * _ROWS_W):
            j = int(_INV[s])
            if j < _B:
                tasks.append((tbl, s, deqt, j))
            else:
                tasks.append((tbl, s, newt, j))
        for r in range(wi * _HEAD_W, (wi + 1) * _HEAD_W):
            tasks.append((batch, r, newt, r))

        n = len(tasks)
        hl = [None] * n
        hw = [None] * n
        for i in range(n + 1):
            if i < n:
                if i >= 2:
                    hw[i - 2].wait()
                src_ref, s, _, _ = tasks[i]
                hl[i] = pltpu.async_copy(src_ref.at[s],
                                         slots.at[sid, i % 2], lsem)
            if i >= 1:
                _, _, dst_ref, j = tasks[i - 1]
                hl[i - 1].wait()
                hw[i - 1] = pltpu.async_copy(slots.at[sid, (i - 1) % 2],
                                             dst_ref.at[j], wsem)
        hw[n - 2].wait()
        hw[n - 1].wait()

    for wi in range(_NW):
        @pl.when(w == wi)
        def _(wi=wi):
            worker_prog(wi)


@functools.partial(
    pl.kernel,
    out_type=[
        jax.ShapeDtypeStruct((_Q, _KDP), jnp.float32),
        jax.ShapeDtypeStruct((_B, _KDP), jnp.float32),
    ],
    mesh=_mesh,
    scratch_types=[
        pltpu.VMEM((16,), jnp.int32),
        pltpu.VMEM((16, _KDP), jnp.float32),
        pltpu.SemaphoreType.DMA,
    ],
)
def _sc_ker_stream(kidx, ker2, lr2, newker2, deqker2, kidx_v, kbuf, hsem):
    w = lax.axis_index("s") * 2 + lax.axis_index("c")

    h0 = pltpu.async_copy(lr2.at[pl.ds(w * 2, 2)],
                          newker2.at[pl.ds(w * 2, 2)], hsem)

    def ker_chunk(idx_off, dst, dst_off):
        pltpu.sync_copy(kidx.at[pl.ds(idx_off, 16)], kidx_v)
        pltpu.async_copy(ker2.at[kidx_v], kbuf, hsem).wait()
        pltpu.sync_copy(kbuf, dst.at[pl.ds(dst_off, 16)])

    @pl.when(w < _KA_W)
    def _():
        ker_chunk(w * 16, newker2, _B + w * 16)

    @pl.when(w >= _KA_W)
    def _():
        ker_chunk((_Q - _B) + (w - _KA_W) * 16, deqker2, (w - _KA_W) * 16)

    h0.wait()


_L = 4
_T = (_Q + _B) // _L
_PH1, _PH2 = _B // _L, 2 * _B // _L

_TCMAPS = np.zeros((7, _T), np.int32)
for _t in range(_T):
    if _t < _PH1:
        for _l in range(_L):
            _TCMAPS[_l, _t] = _IDX[_t * _L + _l]
        _TCMAPS[6, _t] = _t
    elif _t < _PH2:
        _TCMAPS[4, _t] = _t - _PH1
        _TCMAPS[5, _t] = _t - _PH1
    else:
        for _l in range(_L):
            _TCMAPS[_l, _t] = _IDX[_B + (_t - _PH2) * _L + _l]
        _TCMAPS[5, _t] = _t - _PH1
for _l in range(_L):
    _TCMAPS[_l, _PH1:_PH2] = _TCMAPS[_l, _PH2]
_TCMAPS[4, :_PH1] = 0
_TCMAPS[4, _PH2:] = _PH2 - _PH1 - 1
_TCMAPS[5, :_PH1] = 0
_TCMAPS[6, _PH1:] = _PH1 - 1


def _tc_body(maps_ref, t0, t1, t2, t3, batch_ref, new_ref, deq_ref):
    t = pl.program_id(0)
    tbls = (t0, t1, t2, t3)

    @pl.when(t < _PH1)
    def _():
        for l in range(_L):
            deq_ref[pl.ds(l, 1)] = tbls[l][...]

    @pl.when(jnp.logical_and(t >= _PH1, t < _PH2))
    def _():
        new_ref[...] = batch_ref[...]

    @pl.when(t >= _PH2)
    def _():
        for l in range(_L):
            new_ref[pl.ds(l, 1)] = tbls[l][...]


def _map1(kind):
    return pl.BlockSpec((1, _C, _H, _W),
                        lambda t, m, k=kind: (m[k, t], 0, 0, 0))


def _map4(kind):
    return pl.BlockSpec((_L, _C, _H, _W),
                        lambda t, m, k=kind: (m[k, t], 0, 0, 0))


_tc_img_stream = pl.pallas_call(
    _tc_body,
    grid_spec=pltpu.PrefetchScalarGridSpec(
        num_scalar_prefetch=1,
        grid=(_T,),
        in_specs=[_map1(0), _map1(1), _map1(2), _map1(3), _map4(4)],
        out_specs=[_map4(5), _map4(6)],
    ),
    out_shape=[
        jax.ShapeDtypeStruct((_Q, _C, _H, _W), jnp.float32),
        jax.ShapeDtypeStruct((_B, _C, _H, _W), jnp.float32),
    ],
)


def kernel(query, key_img, lr_gt_kernel, queue_q, queue_k, queue_ker):
    ker2 = jnp.pad(queue_ker.reshape(_Q, _KD), ((0, 0), (0, _KDP - _KD)))
    lr2 = jnp.pad(lr_gt_kernel.reshape(_B, _KD), ((0, 0), (0, _KDP - _KD)))
    kidx = jnp.asarray(_KIDX)
    tcmaps = jnp.asarray(_TCMAPS)

    new_qk, k_deq = _sc_img_stream(queue_k, key_img)
    newker2, deqker2 = _sc_ker_stream(kidx, ker2, lr2)
    new_qq, q_deq = _tc_img_stream(tcmaps, queue_q, queue_q, queue_q,
                                   queue_q, query)

    new_qker = newker2[:, :_KD].reshape(_Q, 1, _K, _K)
    ker_deq = deqker2[:, :_KD].reshape(_B, 1, _K, _K)
    return (q_deq, k_deq, ker_deq, new_qq, new_qk, new_qker)


# speedup vs baseline: 2.0594x; 1.0497x over previous
import functools

import jax
import jax.numpy as jnp
import numpy as np
from jax import lax
from jax.experimental import pallas as pl
from jax.experimental.pallas import tpu as pltpu
from jax.experimental.pallas import tpu_sc as plsc

_B = 64
_C = 3
_H = 128
_W = 128
_Q = 512
_K = 21

_D = _C * _H * _W
_KD = 441
_KDP = 512

_NW = 32
_KA_W = (_Q - _B) // 16

_IDX = np.array([
    121, 480, 35, 130, 263, 148, 197, 410, 398, 45, 176, 462, 446, 366, 257,
    179, 139, 315, 501, 188, 312, 499, 318, 448, 304, 99, 309, 144, 152, 189,
    487, 325, 31, 112, 495, 356, 493, 507, 268, 429, 409, 85, 63, 117, 417,
    174, 441, 509, 481, 272, 114, 254, 82, 65, 7, 350, 4, 101, 463, 452, 444,
    102, 78, 163, 157, 302, 183, 29, 240, 177, 278, 259, 108, 305, 83, 129,
    367, 212, 277, 504, 300, 44, 211, 16, 58, 123, 37, 336, 111, 19, 61, 447,
    2, 142, 34, 369, 339, 156, 436, 5, 461, 415, 90, 363, 175, 167, 284, 379,
    251, 110, 72, 155, 178, 323, 291, 388, 269, 354, 368, 219, 510, 153, 30,
    275, 42, 186, 342, 406, 468, 439, 307, 256, 419, 246, 3, 362, 380, 327,
    393, 70, 378, 400, 271, 488, 311, 67, 273, 223, 422, 39, 56, 274, 192,
    169, 349, 218, 195, 476, 173, 245, 241, 69, 383, 80, 22, 6, 321, 199, 345,
    118, 235, 54, 442, 479, 423, 266, 77, 425, 147, 18, 340, 298, 249, 294,
    375, 382, 10, 11, 234, 53, 236, 455, 94, 332, 511, 331, 437, 353, 489,
    287, 32, 217, 283, 355, 407, 159, 440, 15, 470, 184, 49, 137, 50, 138, 20,
    445, 237, 280, 253, 185, 460, 43, 389, 335, 258, 370, 344, 92, 8, 503,
    324, 140, 233, 24, 81, 239, 314, 453, 96, 475, 467, 154, 135, 472, 490,
    469, 500, 264, 160, 106, 128, 265, 426, 386, 191, 9, 200, 40, 187, 71,
    346, 438, 333, 248, 164, 207, 93, 59, 201, 158, 210, 420, 402, 75, 508,
    131, 411, 97, 66, 25, 196, 424, 364, 497, 242, 338, 206, 243, 397, 341,
    450, 414, 238, 295, 432, 431, 308, 73, 320, 13, 52, 491, 203, 289, 303,
    202, 255, 194, 88, 250, 337, 62, 230, 150, 261, 330, 262, 209, 132, 357,
    87, 76, 198, 486, 60, 244, 457, 47, 392, 374, 276, 33, 79, 451, 180, 403,
    247, 14, 459, 286, 421, 458, 228, 17, 38, 86, 231, 190, 232, 482, 23, 105,
    484, 395, 427, 301, 474, 376, 405, 494, 471, 391, 313, 220, 0, 473, 145,
    371, 213, 226, 381, 133, 281, 41, 64, 416, 21, 443, 161, 279, 285, 166,
    124, 116, 449, 26, 165, 168, 193, 57, 208, 181, 89, 146, 182, 126, 125,
    297, 1, 115, 28, 113, 225, 361, 351, 465, 172, 377, 162, 48, 170, 466,
    505, 227, 36, 252, 502, 492, 119, 151, 385, 306, 120, 372, 390, 224, 122,
    270, 100, 418, 433, 329, 365, 396, 91, 222, 55, 496, 498, 103, 51, 293,
    215, 384, 127, 98, 483, 506, 282, 107, 27, 322, 74, 136, 229, 319, 328,
    430, 343, 204, 221, 296, 12, 134, 454, 477, 408, 109, 84, 428, 317, 358,
    394, 299, 205, 171, 288, 143, 68, 267, 216, 435, 149, 485, 434, 141, 464,
    334, 404, 104, 352, 95, 387, 316, 214, 290, 46, 310, 348, 401, 260, 478,
    292, 359, 326, 347, 456, 399, 373, 412, 360, 413], dtype=np.int64)

_KIDX = np.concatenate([_IDX[_B:], _IDX[:_B]]).astype(np.int32)

_INV = np.argsort(_IDX)

_mesh = plsc.VectorSubcoreMesh(core_axis_name="c", subcore_axis_name="s")

_img_out = [
    jax.ShapeDtypeStruct((_Q, _C, _H, _W), jnp.float32),
]

_TAIL_W = (_Q - _B) // _NW
_HEAD_W = _B // _NW


@functools.partial(pl.kernel, out_type=_img_out, mesh=_mesh,
                   scratch_types=[
                       pltpu.VMEM_SHARED((16, 2, _C, _H, _W), jnp.float32),
                       pltpu.SemaphoreType.DMA,
                       pltpu.SemaphoreType.DMA])
def _sc_img_stream(tbl, batch, newt, slots, lsem, wsem):
    w = lax.axis_index("s") * 2 + lax.axis_index("c")

    def worker_prog(wi):
        sid = wi // 2
        tasks = []
        for j in range(_B + wi * _TAIL_W, _B + (wi + 1) * _TAIL_W):
            tasks.append((tbl, int(_IDX[j]), newt, j))
        for r in range(wi * _HEAD_W, (wi + 1) * _HEAD_W):
            tasks.append((batch, r, newt, r))

        n = len(tasks)
        hl = [None] * n
        hw = [None] * n
        for i in range(n + 1):
            if i < n:
                if i >= 2:
                    hw[i - 2].wait()
                src_ref, s, _, _ = tasks[i]
                hl[i] = pltpu.async_copy(src_ref.at[s],
                                         slots.at[sid, i % 2], lsem)
            if i >= 1:
                _, _, dst_ref, j = tasks[i - 1]
                hl[i - 1].wait()
                hw[i - 1] = pltpu.async_copy(slots.at[sid, (i - 1) % 2],
                                             dst_ref.at[j], wsem)
        hw[n - 2].wait()
        hw[n - 1].wait()

    for wi in range(_NW):
        @pl.when(w == wi)
        def _(wi=wi):
            worker_prog(wi)


@functools.partial(
    pl.kernel,
    out_type=[
        jax.ShapeDtypeStruct((_Q, _KDP), jnp.float32),
        jax.ShapeDtypeStruct((_B, _KDP), jnp.float32),
    ],
    mesh=_mesh,
    scratch_types=[
        pltpu.VMEM((16,), jnp.int32),
        pltpu.VMEM((16, _KDP), jnp.float32),
        pltpu.SemaphoreType.DMA,
    ],
)
def _sc_ker_stream(kidx, ker2, lr2, newker2, deqker2, kidx_v, kbuf, hsem):
    w = lax.axis_index("s") * 2 + lax.axis_index("c")

    h0 = pltpu.async_copy(lr2.at[pl.ds(w * 2, 2)],
                          newker2.at[pl.ds(w * 2, 2)], hsem)

    def ker_chunk(idx_off, dst, dst_off):
        pltpu.sync_copy(kidx.at[pl.ds(idx_off, 16)], kidx_v)
        pltpu.async_copy(ker2.at[kidx_v], kbuf, hsem).wait()
        pltpu.sync_copy(kbuf, dst.at[pl.ds(dst_off, 16)])

    @pl.when(w < _KA_W)
    def _():
        ker_chunk(w * 16, newker2, _B + w * 16)

    @pl.when(w >= _KA_W)
    def _():
        ker_chunk((_Q - _B) + (w - _KA_W) * 16, deqker2, (w - _KA_W) * 16)

    h0.wait()


_L = 4
_PH = _B // _L
_T = 2 * _PH

_TCMAPS = np.zeros((10, _T), np.int32)
for _t in range(_T):
    if _t < _PH:
        for _l in range(_L):
            _TCMAPS[_l, _t] = _IDX[_t * _L + _l]
        _TCMAPS[8, _t] = _t
    else:
        for _l in range(_L):
            _TCMAPS[4 + _l, _t] = _IDX[(_t - _PH) * _L + _l]
        _TCMAPS[9, _t] = _t - _PH
for _l in range(_L):
    _TCMAPS[_l, _PH:] = _TCMAPS[_l, _PH - 1]
    _TCMAPS[4 + _l, :_PH] = _TCMAPS[4 + _l, _PH]
_TCMAPS[9, :_PH] = 0
_TCMAPS[8, _PH:] = _PH - 1


def _tc_body(maps_ref, q0, q1, q2, q3, k0, k1, k2, k3, deqq_ref, deqk_ref):
    t = pl.program_id(0)
    qlanes = (q0, q1, q2, q3)
    klanes = (k0, k1, k2, k3)

    @pl.when(t < _PH)
    def _():
        for l in range(_L):
            deqq_ref[pl.ds(l, 1)] = qlanes[l][...]

    @pl.when(t >= _PH)
    def _():
        for l in range(_L):
            deqk_ref[pl.ds(l, 1)] = klanes[l][...]


def _map1(kind):
    return pl.BlockSpec((1, _C, _H, _W),
                        lambda t, m, k=kind: (m[k, t], 0, 0, 0))


def _map4(kind):
    return pl.BlockSpec((_L, _C, _H, _W),
                        lambda t, m, k=kind: (m[k, t], 0, 0, 0))


_tc_deq_stream = pl.pallas_call(
    _tc_body,
    grid_spec=pltpu.PrefetchScalarGridSpec(
        num_scalar_prefetch=1,
        grid=(_T,),
        in_specs=[_map1(k) for k in range(8)],
        out_specs=[_map4(8), _map4(9)],
    ),
    out_shape=[
        jax.ShapeDtypeStruct((_B, _C, _H, _W), jnp.float32),
        jax.ShapeDtypeStruct((_B, _C, _H, _W), jnp.float32),
    ],
)


def kernel(query, key_img, lr_gt_kernel, queue_q, queue_k, queue_ker):
    ker2 = jnp.pad(queue_ker.reshape(_Q, _KD), ((0, 0), (0, _KDP - _KD)))
    lr2 = jnp.pad(lr_gt_kernel.reshape(_B, _KD), ((0, 0), (0, _KDP - _KD)))
    kidx = jnp.asarray(_KIDX)
    tcmaps = jnp.asarray(_TCMAPS)

    (new_qq,) = _sc_img_stream(queue_q, query)
    (new_qk,) = _sc_img_stream(queue_k, key_img)
    newker2, deqker2 = _sc_ker_stream(kidx, ker2, lr2)
    q_deq, k_deq = _tc_deq_stream(tcmaps, queue_q, queue_q, queue_q, queue_q,
                                  queue_k, queue_k, queue_k, queue_k)

    new_qker = newker2[:, :_KD].reshape(_Q, 1, _K, _K)
    ker_deq = deqker2[:, :_KD].reshape(_B, 1, _K, _K)
    return (q_deq, k_deq, ker_deq, new_qq, new_qk, new_qker)


# speedup vs baseline: 2.1077x; 1.0234x over previous
import functools

import jax
import jax.numpy as jnp
import numpy as np
from jax import lax
from jax.experimental import pallas as pl
from jax.experimental.pallas import tpu as pltpu
from jax.experimental.pallas import tpu_sc as plsc

_B = 64
_C = 3
_H = 128
_W = 128
_Q = 512
_K = 21

_D = _C * _H * _W
_KD = 441
_KDP = 512

_NW = 32
_KA_W = (_Q - _B) // 16

_IDX = np.array([
    121, 480, 35, 130, 263, 148, 197, 410, 398, 45, 176, 462, 446, 366, 257,
    179, 139, 315, 501, 188, 312, 499, 318, 448, 304, 99, 309, 144, 152, 189,
    487, 325, 31, 112, 495, 356, 493, 507, 268, 429, 409, 85, 63, 117, 417,
    174, 441, 509, 481, 272, 114, 254, 82, 65, 7, 350, 4, 101, 463, 452, 444,
    102, 78, 163, 157, 302, 183, 29, 240, 177, 278, 259, 108, 305, 83, 129,
    367, 212, 277, 504, 300, 44, 211, 16, 58, 123, 37, 336, 111, 19, 61, 447,
    2, 142, 34, 369, 339, 156, 436, 5, 461, 415, 90, 363, 175, 167, 284, 379,
    251, 110, 72, 155, 178, 323, 291, 388, 269, 354, 368, 219, 510, 153, 30,
    275, 42, 186, 342, 406, 468, 439, 307, 256, 419, 246, 3, 362, 380, 327,
    393, 70, 378, 400, 271, 488, 311, 67, 273, 223, 422, 39, 56, 274, 192,
    169, 349, 218, 195, 476, 173, 245, 241, 69, 383, 80, 22, 6, 321, 199, 345,
    118, 235, 54, 442, 479, 423, 266, 77, 425, 147, 18, 340, 298, 249, 294,
    375, 382, 10, 11, 234, 53, 236, 455, 94, 332, 511, 331, 437, 353, 489,
    287, 32, 217, 283, 355, 407, 159, 440, 15, 470, 184, 49, 137, 50, 138, 20,
    445, 237, 280, 253, 185, 460, 43, 389, 335, 258, 370, 344, 92, 8, 503,
    324, 140, 233, 24, 81, 239, 314, 453, 96, 475, 467, 154, 135, 472, 490,
    469, 500, 264, 160, 106, 128, 265, 426, 386, 191, 9, 200, 40, 187, 71,
    346, 438, 333, 248, 164, 207, 93, 59, 201, 158, 210, 420, 402, 75, 508,
    131, 411, 97, 66, 25, 196, 424, 364, 497, 242, 338, 206, 243, 397, 341,
    450, 414, 238, 295, 432, 431, 308, 73, 320, 13, 52, 491, 203, 289, 303,
    202, 255, 194, 88, 250, 337, 62, 230, 150, 261, 330, 262, 209, 132, 357,
    87, 76, 198, 486, 60, 244, 457, 47, 392, 374, 276, 33, 79, 451, 180, 403,
    247, 14, 459, 286, 421, 458, 228, 17, 38, 86, 231, 190, 232, 482, 23, 105,
    484, 395, 427, 301, 474, 376, 405, 494, 471, 391, 313, 220, 0, 473, 145,
    371, 213, 226, 381, 133, 281, 41, 64, 416, 21, 443, 161, 279, 285, 166,
    124, 116, 449, 26, 165, 168, 193, 57, 208, 181, 89, 146, 182, 126, 125,
    297, 1, 115, 28, 113, 225, 361, 351, 465, 172, 377, 162, 48, 170, 466,
    505, 227, 36, 252, 502, 492, 119, 151, 385, 306, 120, 372, 390, 224, 122,
    270, 100, 418, 433, 329, 365, 396, 91, 222, 55, 496, 498, 103, 51, 293,
    215, 384, 127, 98, 483, 506, 282, 107, 27, 322, 74, 136, 229, 319, 328,
    430, 343, 204, 221, 296, 12, 134, 454, 477, 408, 109, 84, 428, 317, 358,
    394, 299, 205, 171, 288, 143, 68, 267, 216, 435, 149, 485, 434, 141, 464,
    334, 404, 104, 352, 95, 387, 316, 214, 290, 46, 310, 348, 401, 260, 478,
    292, 359, 326, 347, 456, 399, 373, 412, 360, 413], dtype=np.int64)

_KIDX = np.concatenate([_IDX[_B:], _IDX[:_B]]).astype(np.int32)

_INV = np.argsort(_IDX)

_mesh = plsc.VectorSubcoreMesh(core_axis_name="c", subcore_axis_name="s")

_img_out = [
    jax.ShapeDtypeStruct((_Q, _C, _H, _W), jnp.float32),
]

_TAIL_W = (_Q - _B) // _NW
_HEAD_W = _B // _NW


@functools.partial(pl.kernel, out_type=_img_out, mesh=_mesh,
                   scratch_types=[
                       pltpu.VMEM_SHARED((16, 2, _C, _H, _W), jnp.float32),
                       pltpu.SemaphoreType.DMA,
                       pltpu.SemaphoreType.DMA])
def _sc_img_stream(tbl, batch, newt, slots, lsem, wsem):
    w = lax.axis_index("s") * 2 + lax.axis_index("c")

    def worker_prog(wi):
        sid = wi // 2
        tasks = []
        for j in range(_B + wi * _TAIL_W, _B + (wi + 1) * _TAIL_W):
            tasks.append((tbl, int(_IDX[j]), newt, j))
        for r in range(wi * _HEAD_W, (wi + 1) * _HEAD_W):
            tasks.append((batch, r, newt, r))

        n = len(tasks)
        hl = [None] * n
        hw = [None] * n
        for i in range(n + 1):
            if i < n:
                if i >= 2:
                    hw[i - 2].wait()
                src_ref, s, _, _ = tasks[i]
                hl[i] = pltpu.async_copy(src_ref.at[s],
                                         slots.at[sid, i % 2], lsem)
            if i >= 1:
                _, _, dst_ref, j = tasks[i - 1]
                hl[i - 1].wait()
                hw[i - 1] = pltpu.async_copy(slots.at[sid, (i - 1) % 2],
                                             dst_ref.at[j], wsem)
        hw[n - 2].wait()
        hw[n - 1].wait()

    for wi in range(_NW):
        @pl.when(w == wi)
        def _(wi=wi):
            worker_prog(wi)


@functools.partial(
    pl.kernel,
    out_type=[
        jax.ShapeDtypeStruct((_Q, _KDP), jnp.float32),
        jax.ShapeDtypeStruct((_B, _KDP), jnp.float32),
    ],
    mesh=_mesh,
    scratch_types=[
        pltpu.VMEM((16,), jnp.int32),
        pltpu.VMEM((16, _KDP), jnp.float32),
        pltpu.SemaphoreType.DMA,
    ],
)
def _sc_ker_stream(kidx, ker2, lr2, newker2, deqker2, kidx_v, kbuf, hsem):
    w = lax.axis_index("s") * 2 + lax.axis_index("c")

    h0 = pltpu.async_copy(lr2.at[pl.ds(w * 2, 2)],
                          newker2.at[pl.ds(w * 2, 2)], hsem)

    def ker_chunk(idx_off, dst, dst_off):
        pltpu.sync_copy(kidx.at[pl.ds(idx_off, 16)], kidx_v)
        pltpu.async_copy(ker2.at[kidx_v], kbuf, hsem).wait()
        pltpu.sync_copy(kbuf, dst.at[pl.ds(dst_off, 16)])

    @pl.when(w < _KA_W)
    def _():
        ker_chunk(w * 16, newker2, _B + w * 16)

    @pl.when(w >= _KA_W)
    def _():
        ker_chunk((_Q - _B) + (w - _KA_W) * 16, deqker2, (w - _KA_W) * 16)

    h0.wait()


_L = 8
_PH = _B // _L
_T = 2 * _PH

_TCMAPS = np.zeros((2 * _L + 2, _T), np.int32)
for _t in range(_T):
    if _t < _PH:
        for _l in range(_L):
            _TCMAPS[_l, _t] = _IDX[_t * _L + _l]
        _TCMAPS[2 * _L, _t] = _t
    else:
        for _l in range(_L):
            _TCMAPS[_L + _l, _t] = _IDX[(_t - _PH) * _L + _l]
        _TCMAPS[2 * _L + 1, _t] = _t - _PH
for _l in range(_L):
    _TCMAPS[_l, _PH:] = _TCMAPS[_l, _PH - 1]
    _TCMAPS[_L + _l, :_PH] = _TCMAPS[_L + _l, _PH]
_TCMAPS[2 * _L + 1, :_PH] = 0
_TCMAPS[2 * _L, _PH:] = _PH - 1


def _tc_body(maps_ref, *refs):
    qlanes = refs[0:_L]
    klanes = refs[_L:2 * _L]
    deqq_ref, deqk_ref = refs[2 * _L], refs[2 * _L + 1]
    t = pl.program_id(0)

    @pl.when(t < _PH)
    def _():
        for l in range(_L):
            deqq_ref[pl.ds(l, 1)] = qlanes[l][...]

    @pl.when(t >= _PH)
    def _():
        for l in range(_L):
            deqk_ref[pl.ds(l, 1)] = klanes[l][...]


def _map1(kind):
    return pl.BlockSpec((1, _C, _H, _W),
                        lambda t, m, k=kind: (m[k, t], 0, 0, 0))


def _map4(kind):
    return pl.BlockSpec((_L, _C, _H, _W),
                        lambda t, m, k=kind: (m[k, t], 0, 0, 0))


_tc_deq_stream = pl.pallas_call(
    _tc_body,
    grid_spec=pltpu.PrefetchScalarGridSpec(
        num_scalar_prefetch=1,
        grid=(_T,),
        in_specs=[_map1(k) for k in range(2 * _L)],
        out_specs=[_map4(2 * _L), _map4(2 * _L + 1)],
    ),
    out_shape=[
        jax.ShapeDtypeStruct((_B, _C, _H, _W), jnp.float32),
        jax.ShapeDtypeStruct((_B, _C, _H, _W), jnp.float32),
    ],
)


def kernel(query, key_img, lr_gt_kernel, queue_q, queue_k, queue_ker):
    ker2 = jnp.pad(queue_ker.reshape(_Q, _KD), ((0, 0), (0, _KDP - _KD)))
    lr2 = jnp.pad(lr_gt_kernel.reshape(_B, _KD), ((0, 0), (0, _KDP - _KD)))
    kidx = jnp.asarray(_KIDX)
    tcmaps = jnp.asarray(_TCMAPS)

    (new_qq,) = _sc_img_stream(queue_q, query)
    (new_qk,) = _sc_img_stream(queue_k, key_img)
    newker2, deqker2 = _sc_ker_stream(kidx, ker2, lr2)
    q_deq, k_deq = _tc_deq_stream(tcmaps, *([queue_q] * _L), *([queue_k] * _L))

    new_qker = newker2[:, :_KD].reshape(_Q, 1, _K, _K)
    ker_deq = deqker2[:, :_KD].reshape(_B, 1, _K, _K)
    return (q_deq, k_deq, ker_deq, new_qq, new_qk, new_qker)


# speedup vs baseline: 2.2213x; 1.0539x over previous
import functools

import jax
import jax.numpy as jnp
import numpy as np
from jax import lax
from jax.experimental import pallas as pl
from jax.experimental.pallas import tpu as pltpu
from jax.experimental.pallas import tpu_sc as plsc

_B = 64
_C = 3
_H = 128
_W = 128
_Q = 512
_K = 21

_D = _C * _H * _W
_KD = 441
_KDP = 512

_NW = 32
_KA_W = (_Q - _B) // 16

_IDX = np.array([
    121, 480, 35, 130, 263, 148, 197, 410, 398, 45, 176, 462, 446, 366, 257,
    179, 139, 315, 501, 188, 312, 499, 318, 448, 304, 99, 309, 144, 152, 189,
    487, 325, 31, 112, 495, 356, 493, 507, 268, 429, 409, 85, 63, 117, 417,
    174, 441, 509, 481, 272, 114, 254, 82, 65, 7, 350, 4, 101, 463, 452, 444,
    102, 78, 163, 157, 302, 183, 29, 240, 177, 278, 259, 108, 305, 83, 129,
    367, 212, 277, 504, 300, 44, 211, 16, 58, 123, 37, 336, 111, 19, 61, 447,
    2, 142, 34, 369, 339, 156, 436, 5, 461, 415, 90, 363, 175, 167, 284, 379,
    251, 110, 72, 155, 178, 323, 291, 388, 269, 354, 368, 219, 510, 153, 30,
    275, 42, 186, 342, 406, 468, 439, 307, 256, 419, 246, 3, 362, 380, 327,
    393, 70, 378, 400, 271, 488, 311, 67, 273, 223, 422, 39, 56, 274, 192,
    169, 349, 218, 195, 476, 173, 245, 241, 69, 383, 80, 22, 6, 321, 199, 345,
    118, 235, 54, 442, 479, 423, 266, 77, 425, 147, 18, 340, 298, 249, 294,
    375, 382, 10, 11, 234, 53, 236, 455, 94, 332, 511, 331, 437, 353, 489,
    287, 32, 217, 283, 355, 407, 159, 440, 15, 470, 184, 49, 137, 50, 138, 20,
    445, 237, 280, 253, 185, 460, 43, 389, 335, 258, 370, 344, 92, 8, 503,
    324, 140, 233, 24, 81, 239, 314, 453, 96, 475, 467, 154, 135, 472, 490,
    469, 500, 264, 160, 106, 128, 265, 426, 386, 191, 9, 200, 40, 187, 71,
    346, 438, 333, 248, 164, 207, 93, 59, 201, 158, 210, 420, 402, 75, 508,
    131, 411, 97, 66, 25, 196, 424, 364, 497, 242, 338, 206, 243, 397, 341,
    450, 414, 238, 295, 432, 431, 308, 73, 320, 13, 52, 491, 203, 289, 303,
    202, 255, 194, 88, 250, 337, 62, 230, 150, 261, 330, 262, 209, 132, 357,
    87, 76, 198, 486, 60, 244, 457, 47, 392, 374, 276, 33, 79, 451, 180, 403,
    247, 14, 459, 286, 421, 458, 228, 17, 38, 86, 231, 190, 232, 482, 23, 105,
    484, 395, 427, 301, 474, 376, 405, 494, 471, 391, 313, 220, 0, 473, 145,
    371, 213, 226, 381, 133, 281, 41, 64, 416, 21, 443, 161, 279, 285, 166,
    124, 116, 449, 26, 165, 168, 193, 57, 208, 181, 89, 146, 182, 126, 125,
    297, 1, 115, 28, 113, 225, 361, 351, 465, 172, 377, 162, 48, 170, 466,
    505, 227, 36, 252, 502, 492, 119, 151, 385, 306, 120, 372, 390, 224, 122,
    270, 100, 418, 433, 329, 365, 396, 91, 222, 55, 496, 498, 103, 51, 293,
    215, 384, 127, 98, 483, 506, 282, 107, 27, 322, 74, 136, 229, 319, 328,
    430, 343, 204, 221, 296, 12, 134, 454, 477, 408, 109, 84, 428, 317, 358,
    394, 299, 205, 171, 288, 143, 68, 267, 216, 435, 149, 485, 434, 141, 464,
    334, 404, 104, 352, 95, 387, 316, 214, 290, 46, 310, 348, 401, 260, 478,
    292, 359, 326, 347, 456, 399, 373, 412, 360, 413], dtype=np.int64)

_KIDX = np.concatenate([_IDX[_B:], _IDX[:_B]]).astype(np.int32)

_INV = np.argsort(_IDX)

_mesh = plsc.VectorSubcoreMesh(core_axis_name="c", subcore_axis_name="s")

_img_out = [
    jax.ShapeDtypeStruct((_Q, _C, _H, _W), jnp.float32),
]

_TAIL_W = (_Q - _B) // _NW
_HEAD_W = _B // _NW


@functools.partial(pl.kernel, out_type=_img_out, mesh=_mesh,
                   scratch_types=[
                       pltpu.VMEM_SHARED((16, 2, _C, _H, _W), jnp.float32),
                       pltpu.SemaphoreType.DMA,
                       pltpu.SemaphoreType.DMA])
def _sc_img_stream(tbl, batch, newt, slots, lsem, wsem):
    w = lax.axis_index("s") * 2 + lax.axis_index("c")

    def worker_prog(wi):
        sid = wi // 2
        tasks = []
        for j in range(_B + wi * _TAIL_W, _B + (wi + 1) * _TAIL_W):
            tasks.append((tbl, int(_IDX[j]), newt, j))
        for r in range(wi * _HEAD_W, (wi + 1) * _HEAD_W):
            tasks.append((batch, r, newt, r))

        n = len(tasks)
        hl = [None] * n
        hw = [None] * n
        for i in range(n + 1):
            if i < n:
                if i >= 2:
                    hw[i - 2].wait()
                src_ref, s, _, _ = tasks[i]
                hl[i] = pltpu.async_copy(src_ref.at[s],
                                         slots.at[sid, i % 2], lsem)
            if i >= 1:
                _, _, dst_ref, j = tasks[i - 1]
                hl[i - 1].wait()
                hw[i - 1] = pltpu.async_copy(slots.at[sid, (i - 1) % 2],
                                             dst_ref.at[j], wsem)
        hw[n - 2].wait()
        hw[n - 1].wait()

    for wi in range(_NW):
        @pl.when(w == wi)
        def _(wi=wi):
            worker_prog(wi)


@functools.partial(
    pl.kernel,
    out_type=[
        jax.ShapeDtypeStruct((_Q, _KDP), jnp.float32),
        jax.ShapeDtypeStruct((_B, _KDP), jnp.float32),
    ],
    mesh=_mesh,
    scratch_types=[
        pltpu.VMEM((16,), jnp.int32),
        pltpu.VMEM((16, _KDP), jnp.float32),
        pltpu.SemaphoreType.DMA,
    ],
)
def _sc_ker_stream(kidx, ker2, lr2, newker2, deqker2, kidx_v, kbuf, hsem):
    w = lax.axis_index("s") * 2 + lax.axis_index("c")

    h0 = pltpu.async_copy(lr2.at[pl.ds(w * 2, 2)],
                          newker2.at[pl.ds(w * 2, 2)], hsem)

    def ker_chunk(idx_off, dst, dst_off):
        pltpu.sync_copy(kidx.at[pl.ds(idx_off, 16)], kidx_v)
        pltpu.async_copy(ker2.at[kidx_v], kbuf, hsem).wait()
        pltpu.sync_copy(kbuf, dst.at[pl.ds(dst_off, 16)])

    @pl.when(w < _KA_W)
    def _():
        ker_chunk(w * 16, newker2, _B + w * 16)

    @pl.when(w >= _KA_W)
    def _():
        ker_chunk((_Q - _B) + (w - _KA_W) * 16, deqker2, (w - _KA_W) * 16)

    h0.wait()


_L = 8
_P1 = _B // _L
_P2 = 2 * _P1
_P3 = 3 * _P1
_T = _P3 + (_Q - _B) // _L

_MB, _MDQ, _MDK, _MNK = 2 * _L, 2 * _L + 1, 2 * _L + 2, 2 * _L + 3
_TCMAPS = np.zeros((2 * _L + 4, _T), np.int32)
for _t in range(_T):
    if _t < _P1:
        for _l in range(_L):
            _TCMAPS[_l, _t] = _IDX[_t * _L + _l]
        _TCMAPS[_MDQ, _t] = _t
    elif _t < _P2:
        for _l in range(_L):
            _TCMAPS[_L + _l, _t] = _IDX[(_t - _P1) * _L + _l]
        _TCMAPS[_MDK, _t] = _t - _P1
    elif _t < _P3:
        _TCMAPS[_MB, _t] = _t - _P2
        _TCMAPS[_MNK, _t] = _t - _P2
    else:
        for _l in range(_L):
            _TCMAPS[_L + _l, _t] = _IDX[_B + (_t - _P3) * _L + _l]
        _TCMAPS[_MNK, _t] = _P1 + (_t - _P3)
for _l in range(_L):
    _TCMAPS[_l, _P1:] = _TCMAPS[_l, _P1 - 1]
    _TCMAPS[_L + _l, :_P1] = _TCMAPS[_L + _l, _P1]
    _TCMAPS[_L + _l, _P2:_P3] = _TCMAPS[_L + _l, _P3]
_TCMAPS[_MB, :_P2] = 0
_TCMAPS[_MB, _P3:] = _P1 - 1
_TCMAPS[_MDQ, _P1:] = _P1 - 1
_TCMAPS[_MDK, :_P1] = 0
_TCMAPS[_MDK, _P2:] = _P1 - 1
_TCMAPS[_MNK, :_P2] = 0


def _tc_body(maps_ref, *refs):
    qlanes = refs[0:_L]
    klanes = refs[_L:2 * _L]
    batch_ref = refs[_MB]
    deqq_ref, deqk_ref, newk_ref = refs[_MDQ], refs[_MDK], refs[_MNK]
    t = pl.program_id(0)

    @pl.when(t < _P1)
    def _():
        for l in range(_L):
            deqq_ref[pl.ds(l, 1)] = qlanes[l][...]

    @pl.when(jnp.logical_and(t >= _P1, t < _P2))
    def _():
        for l in range(_L):
            deqk_ref[pl.ds(l, 1)] = klanes[l][...]

    @pl.when(jnp.logical_and(t >= _P2, t < _P3))
    def _():
        newk_ref[...] = batch_ref[...]

    @pl.when(t >= _P3)
    def _():
        for l in range(_L):
            newk_ref[pl.ds(l, 1)] = klanes[l][...]


def _map1(kind):
    return pl.BlockSpec((1, _C, _H, _W),
                        lambda t, m, k=kind: (m[k, t], 0, 0, 0))


def _map4(kind):
    return pl.BlockSpec((_L, _C, _H, _W),
                        lambda t, m, k=kind: (m[k, t], 0, 0, 0))


_tc_k_stream = pl.pallas_call(
    _tc_body,
    grid_spec=pltpu.PrefetchScalarGridSpec(
        num_scalar_prefetch=1,
        grid=(_T,),
        in_specs=[_map1(k) for k in range(2 * _L)] + [_map4(_MB)],
        out_specs=[_map4(_MDQ), _map4(_MDK), _map4(_MNK)],
    ),
    out_shape=[
        jax.ShapeDtypeStruct((_B, _C, _H, _W), jnp.float32),
        jax.ShapeDtypeStruct((_B, _C, _H, _W), jnp.float32),
        jax.ShapeDtypeStruct((_Q, _C, _H, _W), jnp.float32),
    ],
)


def kernel(query, key_img, lr_gt_kernel, queue_q, queue_k, queue_ker):
    ker2 = jnp.pad(queue_ker.reshape(_Q, _KD), ((0, 0), (0, _KDP - _KD)))
    lr2 = jnp.pad(lr_gt_kernel.reshape(_B, _KD), ((0, 0), (0, _KDP - _KD)))
    kidx = jnp.asarray(_KIDX)
    tcmaps = jnp.asarray(_TCMAPS)

    (new_qq,) = _sc_img_stream(queue_q, query)
    newker2, deqker2 = _sc_ker_stream(kidx, ker2, lr2)
    q_deq, k_deq, new_qk = _tc_k_stream(tcmaps, *([queue_q] * _L),
                                        *([queue_k] * _L), key_img)

    new_qker = newker2[:, :_KD].reshape(_Q, 1, _K, _K)
    ker_deq = deqker2[:, :_KD].reshape(_B, 1, _K, _K)
    return (q_deq, k_deq, ker_deq, new_qq, new_qk, new_qker)


# speedup vs baseline: 2.2830x; 1.0277x over previous
import functools

import jax
import jax.numpy as jnp
import numpy as np
from jax import lax
from jax.experimental import pallas as pl
from jax.experimental.pallas import tpu as pltpu
from jax.experimental.pallas import tpu_sc as plsc

_B = 64
_C = 3
_H = 128
_W = 128
_Q = 512
_K = 21

_D = _C * _H * _W
_KD = 441
_KDP = 512

_NW = 32
_KA_W = (_Q - _B) // 16

_IDX = np.array([
    121, 480, 35, 130, 263, 148, 197, 410, 398, 45, 176, 462, 446, 366, 257,
    179, 139, 315, 501, 188, 312, 499, 318, 448, 304, 99, 309, 144, 152, 189,
    487, 325, 31, 112, 495, 356, 493, 507, 268, 429, 409, 85, 63, 117, 417,
    174, 441, 509, 481, 272, 114, 254, 82, 65, 7, 350, 4, 101, 463, 452, 444,
    102, 78, 163, 157, 302, 183, 29, 240, 177, 278, 259, 108, 305, 83, 129,
    367, 212, 277, 504, 300, 44, 211, 16, 58, 123, 37, 336, 111, 19, 61, 447,
    2, 142, 34, 369, 339, 156, 436, 5, 461, 415, 90, 363, 175, 167, 284, 379,
    251, 110, 72, 155, 178, 323, 291, 388, 269, 354, 368, 219, 510, 153, 30,
    275, 42, 186, 342, 406, 468, 439, 307, 256, 419, 246, 3, 362, 380, 327,
    393, 70, 378, 400, 271, 488, 311, 67, 273, 223, 422, 39, 56, 274, 192,
    169, 349, 218, 195, 476, 173, 245, 241, 69, 383, 80, 22, 6, 321, 199, 345,
    118, 235, 54, 442, 479, 423, 266, 77, 425, 147, 18, 340, 298, 249, 294,
    375, 382, 10, 11, 234, 53, 236, 455, 94, 332, 511, 331, 437, 353, 489,
    287, 32, 217, 283, 355, 407, 159, 440, 15, 470, 184, 49, 137, 50, 138, 20,
    445, 237, 280, 253, 185, 460, 43, 389, 335, 258, 370, 344, 92, 8, 503,
    324, 140, 233, 24, 81, 239, 314, 453, 96, 475, 467, 154, 135, 472, 490,
    469, 500, 264, 160, 106, 128, 265, 426, 386, 191, 9, 200, 40, 187, 71,
    346, 438, 333, 248, 164, 207, 93, 59, 201, 158, 210, 420, 402, 75, 508,
    131, 411, 97, 66, 25, 196, 424, 364, 497, 242, 338, 206, 243, 397, 341,
    450, 414, 238, 295, 432, 431, 308, 73, 320, 13, 52, 491, 203, 289, 303,
    202, 255, 194, 88, 250, 337, 62, 230, 150, 261, 330, 262, 209, 132, 357,
    87, 76, 198, 486, 60, 244, 457, 47, 392, 374, 276, 33, 79, 451, 180, 403,
    247, 14, 459, 286, 421, 458, 228, 17, 38, 86, 231, 190, 232, 482, 23, 105,
    484, 395, 427, 301, 474, 376, 405, 494, 471, 391, 313, 220, 0, 473, 145,
    371, 213, 226, 381, 133, 281, 41, 64, 416, 21, 443, 161, 279, 285, 166,
    124, 116, 449, 26, 165, 168, 193, 57, 208, 181, 89, 146, 182, 126, 125,
    297, 1, 115, 28, 113, 225, 361, 351, 465, 172, 377, 162, 48, 170, 466,
    505, 227, 36, 252, 502, 492, 119, 151, 385, 306, 120, 372, 390, 224, 122,
    270, 100, 418, 433, 329, 365, 396, 91, 222, 55, 496, 498, 103, 51, 293,
    215, 384, 127, 98, 483, 506, 282, 107, 27, 322, 74, 136, 229, 319, 328,
    430, 343, 204, 221, 296, 12, 134, 454, 477, 408, 109, 84, 428, 317, 358,
    394, 299, 205, 171, 288, 143, 68, 267, 216, 435, 149, 485, 434, 141, 464,
    334, 404, 104, 352, 95, 387, 316, 214, 290, 46, 310, 348, 401, 260, 478,
    292, 359, 326, 347, 456, 399, 373, 412, 360, 413], dtype=np.int64)

_KIDX = np.concatenate([_IDX[_B:], _IDX[:_B]]).astype(np.int32)

_INV = np.argsort(_IDX)

_mesh = plsc.VectorSubcoreMesh(core_axis_name="c", subcore_axis_name="s")

_img_out = [
    jax.ShapeDtypeStruct((_Q, _C, _H, _W), jnp.float32),
    jax.ShapeDtypeStruct((_B, _C, _H, _W), jnp.float32),
]

_ROWS_W = _Q // _NW
_HEAD_W = _B // _NW


@functools.partial(pl.kernel, out_type=_img_out, mesh=_mesh,
                   scratch_types=[
                       pltpu.VMEM_SHARED((16, 2, _C, _H, _W), jnp.float32),
                       pltpu.SemaphoreType.DMA,
                       pltpu.SemaphoreType.DMA])
def _sc_img_stream(tbl, batch, newt, deqt, slots, lsem, wsem):
    w = lax.axis_index("s") * 2 + lax.axis_index("c")

    def worker_prog(wi):
        sid = wi // 2
        tasks = []
        for s in range(wi * _ROWS_W, (wi + 1) * _ROWS_W):
            j = int(_INV[s])
            if j < _B:
                tasks.append((tbl, s, deqt, j))
            else:
                tasks.append((tbl, s, newt, j))
        for r in range(wi * _HEAD_W, (wi + 1) * _HEAD_W):
            tasks.append((batch, r, newt, r))

        n = len(tasks)
        hl = [None] * n
        hw = [None] * n
        for i in range(n + 1):
            if i < n:
                if i >= 2:
                    hw[i - 2].wait()
                src_ref, s, _, _ = tasks[i]
                hl[i] = pltpu.async_copy(src_ref.at[s],
                                         slots.at[sid, i % 2], lsem)
            if i >= 1:
                _, _, dst_ref, j = tasks[i - 1]
                hl[i - 1].wait()
                hw[i - 1] = pltpu.async_copy(slots.at[sid, (i - 1) % 2],
                                             dst_ref.at[j], wsem)
        hw[n - 2].wait()
        hw[n - 1].wait()

    for wi in range(_NW):
        @pl.when(w == wi)
        def _(wi=wi):
            worker_prog(wi)


@functools.partial(
    pl.kernel,
    out_type=[
        jax.ShapeDtypeStruct((_Q, _KDP), jnp.float32),
        jax.ShapeDtypeStruct((_B, _KDP), jnp.float32),
    ],
    mesh=_mesh,
    scratch_types=[
        pltpu.VMEM((16,), jnp.int32),
        pltpu.VMEM((16, _KDP), jnp.float32),
        pltpu.SemaphoreType.DMA,
    ],
)
def _sc_ker_stream(kidx, ker2, lr2, newker2, deqker2, kidx_v, kbuf, hsem):
    w = lax.axis_index("s") * 2 + lax.axis_index("c")

    h0 = pltpu.async_copy(lr2.at[pl.ds(w * 2, 2)],
                          newker2.at[pl.ds(w * 2, 2)], hsem)

    def ker_chunk(idx_off, dst, dst_off):
        pltpu.sync_copy(kidx.at[pl.ds(idx_off, 16)], kidx_v)
        pltpu.async_copy(ker2.at[kidx_v], kbuf, hsem).wait()
        pltpu.sync_copy(kbuf, dst.at[pl.ds(dst_off, 16)])

    @pl.when(w < _KA_W)
    def _():
        ker_chunk(w * 16, newker2, _B + w * 16)

    @pl.when(w >= _KA_W)
    def _():
        ker_chunk((_Q - _B) + (w - _KA_W) * 16, deqker2, (w - _KA_W) * 16)

    h0.wait()


_L = 8
_P1 = _B // _L
_P2 = 2 * _P1
_T = _P2 + (_Q - _B) // _L

_MB, _MDK, _MNK = _L, _L + 1, _L + 2
_TCMAPS = np.zeros((_L + 3, _T), np.int32)
for _t in range(_T):
    if _t < _P1:
        for _l in range(_L):
            _TCMAPS[_l, _t] = _IDX[_t * _L + _l]
        _TCMAPS[_MDK, _t] = _t
    elif _t < _P2:
        _TCMAPS[_MB, _t] = _t - _P1
        _TCMAPS[_MNK, _t] = _t - _P1
    else:
        for _l in range(_L):
            _TCMAPS[_l, _t] = _IDX[_B + (_t - _P2) * _L + _l]
        _TCMAPS[_MNK, _t] = _P1 + (_t - _P2)
for _l in range(_L):
    _TCMAPS[_l, _P1:_P2] = _TCMAPS[_l, _P2]
_TCMAPS[_MB, :_P1] = 0
_TCMAPS[_MB, _P2:] = _P1 - 1
_TCMAPS[_MDK, _P1:] = _P1 - 1
_TCMAPS[_MNK, :_P1] = 0


def _tc_body(maps_ref, *refs):
    klanes = refs[0:_L]
    batch_ref = refs[_MB]
    deqk_ref, newk_ref = refs[_MDK], refs[_MNK]
    t = pl.program_id(0)

    @pl.when(t < _P1)
    def _():
        for l in range(_L):
            deqk_ref[pl.ds(l, 1)] = klanes[l][...]

    @pl.when(jnp.logical_and(t >= _P1, t < _P2))
    def _():
        newk_ref[...] = batch_ref[...]

    @pl.when(t >= _P2)
    def _():
        for l in range(_L):
            newk_ref[pl.ds(l, 1)] = klanes[l][...]


def _map1(kind):
    return pl.BlockSpec((1, _C, _H, _W),
                        lambda t, m, k=kind: (m[k, t], 0, 0, 0))


def _map4(kind):
    return pl.BlockSpec((_L, _C, _H, _W),
                        lambda t, m, k=kind: (m[k, t], 0, 0, 0))


_tc_k_stream = pl.pallas_call(
    _tc_body,
    grid_spec=pltpu.PrefetchScalarGridSpec(
        num_scalar_prefetch=1,
        grid=(_T,),
        in_specs=[_map1(k) for k in range(_L)] + [_map4(_MB)],
        out_specs=[_map4(_MDK), _map4(_MNK)],
    ),
    out_shape=[
        jax.ShapeDtypeStruct((_B, _C, _H, _W), jnp.float32),
        jax.ShapeDtypeStruct((_Q, _C, _H, _W), jnp.float32),
    ],
)


def kernel(query, key_img, lr_gt_kernel, queue_q, queue_k, queue_ker):
    ker2 = jnp.pad(queue_ker.reshape(_Q, _KD), ((0, 0), (0, _KDP - _KD)))
    lr2 = jnp.pad(lr_gt_kernel.reshape(_B, _KD), ((0, 0), (0, _KDP - _KD)))
    kidx = jnp.asarray(_KIDX)
    tcmaps = jnp.asarray(_TCMAPS)

    new_qq, q_deq = _sc_img_stream(queue_q, query)
    newker2, deqker2 = _sc_ker_stream(kidx, ker2, lr2)
    k_deq, new_qk = _tc_k_stream(tcmaps, *([queue_k] * _L), key_img)

    new_qker = newker2[:, :_KD].reshape(_Q, 1, _K, _K)
    ker_deq = deqker2[:, :_KD].reshape(_B, 1, _K, _K)
    return (q_deq, k_deq, ker_deq, new_qq, new_qk, new_qker)
